# baseline (speedup 1.0000x reference)
import functools

import jax
import jax.numpy as jnp
from jax.experimental import pallas as pl

N = 100000
E = 1600000
L = 3
D = 32

_BN = 1000
_BE = 12800


def _node_mlp_body(x_ref, W1a_ref, b1a_ref, W1b_ref, b1b_ref, out_ref):
    t = jnp.maximum(
        jnp.dot(x_ref[...], W1a_ref[...], preferred_element_type=jnp.float32)
        + b1a_ref[...], 0.0)
    out_ref[...] = (
        jnp.dot(t, W1b_ref[...], preferred_element_type=jnp.float32)
        + b1b_ref[...])


def _node_mlp(x, W1a, b1a, W1b, b1b):
    nb = N // _BN
    return pl.pallas_call(
        _node_mlp_body,
        grid=(nb,),
        in_specs=[
            pl.BlockSpec((_BN, 12), lambda i: (i, 0)),
            pl.BlockSpec((12, 27), lambda i: (0, 0)),
            pl.BlockSpec((1, 27), lambda i: (0, 0)),
            pl.BlockSpec((27, 32), lambda i: (0, 0)),
            pl.BlockSpec((1, 32), lambda i: (0, 0)),
        ],
        out_specs=pl.BlockSpec((_BN, 32), lambda i: (i, 0)),
        out_shape=jax.ShapeDtypeStruct((N, 32), jnp.float32),
    )(x, W1a, b1a.reshape(1, 27), W1b, b1b.reshape(1, 32))


def _edge_mlp_body(ea_ref, W2a_ref, b2a_ref, W2b_ref, b2b_ref, out_ref):
    t = jnp.maximum(
        jnp.dot(ea_ref[...], W2a_ref[...], preferred_element_type=jnp.float32)
        + b2a_ref[...], 0.0)
    out_ref[...] = (
        jnp.dot(t, W2b_ref[...], preferred_element_type=jnp.float32)
        + b2b_ref[...])


def _edge_mlp(edge_attr, W2a, b2a, W2b, b2b):
    nb = E // _BE
    return pl.pallas_call(
        _edge_mlp_body,
        grid=(nb,),
        in_specs=[
            pl.BlockSpec((_BE, 3), lambda i: (i, 0)),
            pl.BlockSpec((3, 9), lambda i: (0, 0)),
            pl.BlockSpec((1, 9), lambda i: (0, 0)),
            pl.BlockSpec((9, 32), lambda i: (0, 0)),
            pl.BlockSpec((1, 32), lambda i: (0, 0)),
        ],
        out_specs=pl.BlockSpec((_BE, 32), lambda i: (i, 0)),
        out_shape=jax.ShapeDtypeStruct((E, 32), jnp.float32),
    )(edge_attr, W2a, b2a.reshape(1, 9), W2b, b2b.reshape(1, 32))


def _layer_mlp_body(h_ref, aggr_ref, Wc1_ref, bc1_ref, Wc2_ref, bc2_ref,
                    z_ref, sums_ref):
    i = pl.program_id(0)
    z0 = h_ref[...] + aggr_ref[...]
    t = jnp.maximum(
        jnp.dot(z0, Wc1_ref[...], preferred_element_type=jnp.float32)
        + bc1_ref[...], 0.0)
    z = jnp.dot(t, Wc2_ref[...], preferred_element_type=jnp.float32) + bc2_ref[...]
    z_ref[...] = z
    s = jnp.sum(z, axis=0, keepdims=True)
    s2 = jnp.sum(z * z, axis=0, keepdims=True)
    blk = jnp.concatenate([s, s2], axis=0)

    @pl.when(i == 0)
    def _():
        sums_ref[...] = blk

    @pl.when(i != 0)
    def _():
        sums_ref[...] += blk


def _layer_mlp(h, aggr, Wc1, bc1, Wc2, bc2):
    nb = N // _BN
    return pl.pallas_call(
        _layer_mlp_body,
        grid=(nb,),
        in_specs=[
            pl.BlockSpec((_BN, 32), lambda i: (i, 0)),
            pl.BlockSpec((_BN, 32), lambda i: (i, 0)),
            pl.BlockSpec((32, 64), lambda i: (0, 0)),
            pl.BlockSpec((1, 64), lambda i: (0, 0)),
            pl.BlockSpec((64, 32), lambda i: (0, 0)),
            pl.BlockSpec((1, 32), lambda i: (0, 0)),
        ],
        out_specs=[
            pl.BlockSpec((_BN, 32), lambda i: (i, 0)),
            pl.BlockSpec((2, 32), lambda i: (0, 0)),
        ],
        out_shape=[
            jax.ShapeDtypeStruct((N, 32), jnp.float32),
            jax.ShapeDtypeStruct((2, 32), jnp.float32),
        ],
    )(h, aggr, Wc1, bc1.reshape(1, 64), Wc2, bc2.reshape(1, 32))


def _bn_body(z_ref, scale_ref, shift_ref, out_ref, *, relu):
    o = z_ref[...] * scale_ref[...] + shift_ref[...]
    if relu:
        o = jnp.maximum(o, 0.0)
    out_ref[...] = o


def _bn_apply(z, scale, shift, relu):
    nb = N // _BN
    return pl.pallas_call(
        functools.partial(_bn_body, relu=relu),
        grid=(nb,),
        in_specs=[
            pl.BlockSpec((_BN, 32), lambda i: (i, 0)),
            pl.BlockSpec((1, 32), lambda i: (0, 0)),
            pl.BlockSpec((1, 32), lambda i: (0, 0)),
        ],
        out_specs=pl.BlockSpec((_BN, 32), lambda i: (i, 0)),
        out_shape=jax.ShapeDtypeStruct((N, 32), jnp.float32),
    )(z, scale, shift)


def kernel(x, edge_attr, edge_index, W1a, b1a, W1b, b1b, W2a, b2a, W2b, b2b,
           Wc1, bc1, Wc2, bc2, gamma, beta):
    src = edge_index[0]
    dst = edge_index[1]

    h = _node_mlp(x, W1a, b1a, W1b, b1b)
    ea = _edge_mlp(edge_attr, W2a, b2a, W2b, b2b)

    h_cur = h
    for l in range(L):
        msg = jnp.maximum(h_cur[src] + ea, 0.0)
        aggr = jax.ops.segment_sum(msg, dst, num_segments=N)
        z, sums = _layer_mlp(h_cur, aggr, Wc1[l], bc1[l], Wc2[l], bc2[l])
        mean = sums[0] / N
        var = sums[1] / N - mean * mean
        scale = gamma[l] / jnp.sqrt(var + 1e-5)
        shift = beta[l] - mean * scale
        h_cur = _bn_apply(z, scale.reshape(1, 32), shift.reshape(1, 32),
                          relu=(l != L - 1))
    return h_cur


# speedup vs baseline: 3.1839x; 3.1839x over previous
import functools

import jax
import jax.numpy as jnp
from jax import lax
from jax.experimental import pallas as pl
from jax.experimental.pallas import tpu as pltpu
from jax.experimental.pallas import tpu_sc as plsc

N = 100000
E = 1600000
L = 3
D = 32

NT = 100096
E_PAD = 1605632
ER = E_PAD // 128
EPW = E_PAD // 16
CHUNKS = 196
ZROWS = NT // 16

_BN = 800
_BE = 6272


def _node_mlp_body(x_ref, W1a_ref, b1a_ref, W1b_ref, b1b_ref, out_ref):
    c = pl.program_id(0)
    t = jnp.maximum(
        jnp.dot(x_ref[...], W1a_ref[...], preferred_element_type=jnp.float32)
        + b1a_ref[...], 0.0)
    val = (jnp.dot(t, W1b_ref[...], preferred_element_type=jnp.float32)
           + b1b_ref[...])
    half = jnp.where(c == 0, val[:, :16], val[:, 16:])
    out_ref[...] = half[None]


def _node_mlp(x, W1a, b1a, W1b, b1b):
    return pl.pallas_call(
        _node_mlp_body,
        grid=(2, N // _BN),
        in_specs=[
            pl.BlockSpec((_BN, 12), lambda c, i: (i, 0)),
            pl.BlockSpec((12, 27), lambda c, i: (0, 0)),
            pl.BlockSpec((1, 27), lambda c, i: (0, 0)),
            pl.BlockSpec((27, 32), lambda c, i: (0, 0)),
            pl.BlockSpec((1, 32), lambda c, i: (0, 0)),
        ],
        out_specs=pl.BlockSpec((1, _BN, 16), lambda c, i: (c, i, 0)),
        out_shape=jax.ShapeDtypeStruct((2, NT, 16), jnp.float32),
    )(x, W1a, b1a.reshape(1, 27), W1b, b1b.reshape(1, 32))


def _edge_mlp_body(ea_ref, W2a_ref, b2a_ref, W2b_ref, b2b_ref, out_ref):
    c = pl.program_id(0)
    t = jnp.maximum(
        jnp.dot(ea_ref[...], W2a_ref[...], preferred_element_type=jnp.float32)
        + b2a_ref[...], 0.0)
    val = (jnp.dot(t, W2b_ref[...], preferred_element_type=jnp.float32)
           + b2b_ref[...])
    half = jnp.where(c == 0, val[:, :16], val[:, 16:])
    out_ref[...] = half[None]


def _edge_mlp(edge_attr, W2a, b2a, W2b, b2b):
    return pl.pallas_call(
        _edge_mlp_body,
        grid=(2, E_PAD // _BE),
        in_specs=[
            pl.BlockSpec((_BE, 3), lambda c, i: (i, 0)),
            pl.BlockSpec((3, 9), lambda c, i: (0, 0)),
            pl.BlockSpec((1, 9), lambda c, i: (0, 0)),
            pl.BlockSpec((9, 32), lambda c, i: (0, 0)),
            pl.BlockSpec((1, 32), lambda c, i: (0, 0)),
        ],
        out_specs=pl.BlockSpec((1, _BE, 16), lambda c, i: (c, i, 0)),
        out_shape=jax.ShapeDtypeStruct((2, E_PAD, 16), jnp.float32),
    )(edge_attr, W2a, b2a.reshape(1, 9), W2b, b2b.reshape(1, 32))


def _sc_aggr_body(h2_hbm, ea2_hbm, src_hbm, dst_hbm, out_hbm,
                  aggr_sh, sidx, didx, rows, eab, gsem):
    c = lax.axis_index("c")
    s = lax.axis_index("s")

    def zstore(i, carry):
        rows[i] = jnp.zeros((16,), jnp.float32)
        return carry
    lax.fori_loop(0, 512, zstore, 0)
    for k in range(12):
        pltpu.sync_copy(rows, aggr_sh.at[pl.ds(s * ZROWS + k * 512, 512)])
    pltpu.sync_copy(rows.at[pl.ds(0, 112)],
                    aggr_sh.at[pl.ds(s * ZROWS + 6144, 112)])
    plsc.subcore_barrier()

    def chunk(g, carry):
        rbase = s * (EPW // 128) + g * 4
        ebase = c * E_PAD + s * EPW + g * 512
        pltpu.sync_copy(src_hbm.at[pl.ds(c * ER + rbase, 4)], sidx)
        pltpu.sync_copy(dst_hbm.at[pl.ds(rbase, 4)], didx)
        cps = [
            pltpu.async_copy(h2_hbm.at[sidx.at[j]],
                             rows.at[pl.ds(j * 128, 128)], gsem)
            for j in range(4)
        ]
        pltpu.sync_copy(ea2_hbm.at[pl.ds(ebase, 512)], eab)
        for cp in cps:
            cp.wait()

        def edge(i, cc):
            rows[i] = jnp.maximum(rows[i] + eab[i], 0.0)
            return cc
        lax.fori_loop(0, 512, edge, 0)

        for j in range(4):
            pltpu.sync_copy(rows.at[pl.ds(j * 128, 128)],
                            aggr_sh.at[didx.at[j]], add=True)
        return carry

    lax.fori_loop(0, CHUNKS, chunk, 0)
    plsc.subcore_barrier()

    for k in range(12):
        off = s * ZROWS + k * 512
        pltpu.sync_copy(aggr_sh.at[pl.ds(off, 512)],
                        out_hbm.at[pl.ds(c * NT + off, 512)])
    off = s * ZROWS + 6144
    pltpu.sync_copy(aggr_sh.at[pl.ds(off, 112)],
                    out_hbm.at[pl.ds(c * NT + off, 112)])


def _sc_aggregate(h2_flat, ea2_flat, src2, dstp):
    mesh = plsc.VectorSubcoreMesh(core_axis_name="c", subcore_axis_name="s")
    f = pl.kernel(
        _sc_aggr_body,
        out_type=jax.ShapeDtypeStruct((2 * NT, 16), jnp.float32),
        mesh=mesh,
        compiler_params=pltpu.CompilerParams(use_tc_tiling_on_sc=False),
        scratch_types=[
            pltpu.VMEM_SHARED((NT, 16), jnp.float32),
            pltpu.VMEM((4, 128), jnp.int32),
            pltpu.VMEM((4, 128), jnp.int32),
            pltpu.VMEM((512, 16), jnp.float32),
            pltpu.VMEM((512, 16), jnp.float32),
            pltpu.SemaphoreType.DMA,
        ],
    )
    return f(h2_flat, ea2_flat, src2, dstp)


def _layer_mlp_body(h0_ref, h1_ref, a0_ref, a1_ref, Wc1_ref, bc1_ref,
                    Wc2_ref, bc2_ref, z_ref, sums_ref):
    i = pl.program_id(0)
    z0 = jnp.concatenate(
        [h0_ref[...] + a0_ref[...], h1_ref[...] + a1_ref[...]], axis=1)
    t = jnp.maximum(
        jnp.dot(z0, Wc1_ref[...], preferred_element_type=jnp.float32)
        + bc1_ref[...], 0.0)
    z = jnp.dot(t, Wc2_ref[...], preferred_element_type=jnp.float32) + bc2_ref[...]
    z_ref[...] = z
    blk = jnp.concatenate(
        [jnp.sum(z, axis=0, keepdims=True),
         jnp.sum(z * z, axis=0, keepdims=True)], axis=0)

    @pl.when(i == 0)
    def _():
        sums_ref[...] = blk

    @pl.when(i != 0)
    def _():
        sums_ref[...] += blk


def _layer_mlp(h2_flat, aggr_flat, Wc1, bc1, Wc2, bc2):
    half = pl.BlockSpec((_BN, 16), lambda i: (i, 0))
    return pl.pallas_call(
        _layer_mlp_body,
        grid=(N // _BN,),
        in_specs=[
            half, half, half, half,
            pl.BlockSpec((32, 64), lambda i: (0, 0)),
            pl.BlockSpec((1, 64), lambda i: (0, 0)),
            pl.BlockSpec((64, 32), lambda i: (0, 0)),
            pl.BlockSpec((1, 32), lambda i: (0, 0)),
        ],
        out_specs=[
            pl.BlockSpec((_BN, 32), lambda i: (i, 0)),
            pl.BlockSpec((2, 32), lambda i: (0, 0)),
        ],
        out_shape=[
            jax.ShapeDtypeStruct((N, 32), jnp.float32),
            jax.ShapeDtypeStruct((2, 32), jnp.float32),
        ],
    )(h2_flat[:N], h2_flat[NT:NT + N], aggr_flat[:N], aggr_flat[NT:NT + N],
      Wc1, bc1.reshape(1, 64), Wc2, bc2.reshape(1, 32))


def _bn_mid_body(z_ref, sc_ref, sh_ref, out_ref):
    c = pl.program_id(0)
    val = jnp.maximum(z_ref[...] * sc_ref[...] + sh_ref[...], 0.0)
    half = jnp.where(c == 0, val[:, :16], val[:, 16:])
    out_ref[...] = half[None]


def _bn_mid(z, scale, shift):
    return pl.pallas_call(
        _bn_mid_body,
        grid=(2, N // _BN),
        in_specs=[
            pl.BlockSpec((_BN, 32), lambda c, i: (i, 0)),
            pl.BlockSpec((1, 32), lambda c, i: (0, 0)),
            pl.BlockSpec((1, 32), lambda c, i: (0, 0)),
        ],
        out_specs=pl.BlockSpec((1, _BN, 16), lambda c, i: (c, i, 0)),
        out_shape=jax.ShapeDtypeStruct((2, NT, 16), jnp.float32),
    )(z, scale, shift)


def _bn_final_body(z_ref, sc_ref, sh_ref, out_ref):
    out_ref[...] = z_ref[...] * sc_ref[...] + sh_ref[...]


def _bn_final(z, scale, shift):
    return pl.pallas_call(
        _bn_final_body,
        grid=(N // _BN,),
        in_specs=[
            pl.BlockSpec((_BN, 32), lambda i: (i, 0)),
            pl.BlockSpec((1, 32), lambda i: (0, 0)),
            pl.BlockSpec((1, 32), lambda i: (0, 0)),
        ],
        out_specs=pl.BlockSpec((_BN, 32), lambda i: (i, 0)),
        out_shape=jax.ShapeDtypeStruct((N, 32), jnp.float32),
    )(z, scale, shift)


def kernel(x, edge_attr, edge_index, W1a, b1a, W1b, b1b, W2a, b2a, W2b, b2b,
           Wc1, bc1, Wc2, bc2, gamma, beta):
    src = edge_index[0]
    dst = edge_index[1]

    pad = N + (jnp.arange(E_PAD - E, dtype=jnp.int32) % 16)
    srcp = jnp.concatenate([src, pad])
    dstp = jnp.concatenate([dst, pad]).reshape(ER, 128)
    src2 = (srcp[None, :]
            + jnp.array([0, NT], dtype=jnp.int32)[:, None]).reshape(2 * ER, 128)

    h2 = _node_mlp(x, W1a, b1a, W1b, b1b).reshape(2 * NT, 16)
    ea2 = _edge_mlp(edge_attr, W2a, b2a, W2b, b2b).reshape(2 * E_PAD, 16)

    for l in range(L):
        aggr = _sc_aggregate(h2, ea2, src2, dstp)
        z, sums = _layer_mlp(h2, aggr, Wc1[l], bc1[l], Wc2[l], bc2[l])
        mean = sums[0] / N
        var = sums[1] / N - mean * mean
        scale = (gamma[l] / jnp.sqrt(var + 1e-5)).reshape(1, 32)
        shift = (beta[l] - mean * scale[0]).reshape(1, 32)
        if l != L - 1:
            h2 = _bn_mid(z, scale, shift).reshape(2 * NT, 16)
        else:
            out = _bn_final(z, scale, shift)
    return out


# speedup vs baseline: 4.3044x; 1.3520x over previous
import functools

import jax
import jax.numpy as jnp
from jax import lax
from jax.experimental import pallas as pl
from jax.experimental.pallas import tpu as pltpu
from jax.experimental.pallas import tpu_sc as plsc

N = 100000
E = 1600000
L = 3
D = 32

NT = 100096
E_PAD = 1605632
ER = E_PAD // 128
EPW = E_PAD // 16
CHUNKS = 392
ZROWS = NT // 16

_BN = 800
_BE = 6272


def _node_mlp_body(x_ref, W1a_ref, b1a_ref, W1b_ref, b1b_ref, out_ref):
    c = pl.program_id(0)
    t = jnp.maximum(
        jnp.dot(x_ref[...], W1a_ref[...], preferred_element_type=jnp.float32)
        + b1a_ref[...], 0.0)
    val = (jnp.dot(t, W1b_ref[...], preferred_element_type=jnp.float32)
           + b1b_ref[...])
    half = jnp.where(c == 0, val[:, :16], val[:, 16:])
    out_ref[...] = half[None]


def _node_mlp(x, W1a, b1a, W1b, b1b):
    return pl.pallas_call(
        _node_mlp_body,
        grid=(2, N // _BN),
        in_specs=[
            pl.BlockSpec((_BN, 12), lambda c, i: (i, 0)),
            pl.BlockSpec((12, 27), lambda c, i: (0, 0)),
            pl.BlockSpec((1, 27), lambda c, i: (0, 0)),
            pl.BlockSpec((27, 32), lambda c, i: (0, 0)),
            pl.BlockSpec((1, 32), lambda c, i: (0, 0)),
        ],
        out_specs=pl.BlockSpec((1, _BN, 16), lambda c, i: (c, i, 0)),
        out_shape=jax.ShapeDtypeStruct((2, NT, 16), jnp.float32),
    )(x, W1a, b1a.reshape(1, 27), W1b, b1b.reshape(1, 32))


def _edge_mlp_body(ea_ref, W2a_ref, b2a_ref, W2b_ref, b2b_ref, out_ref):
    c = pl.program_id(0)
    t = jnp.maximum(
        jnp.dot(ea_ref[...], W2a_ref[...], preferred_element_type=jnp.float32)
        + b2a_ref[...], 0.0)
    val = (jnp.dot(t, W2b_ref[...], preferred_element_type=jnp.float32)
           + b2b_ref[...])
    half = jnp.where(c == 0, val[:, :16], val[:, 16:])
    out_ref[...] = half[None]


def _edge_mlp(edge_attr, W2a, b2a, W2b, b2b):
    return pl.pallas_call(
        _edge_mlp_body,
        grid=(2, E_PAD // _BE),
        in_specs=[
            pl.BlockSpec((_BE, 3), lambda c, i: (i, 0)),
            pl.BlockSpec((3, 9), lambda c, i: (0, 0)),
            pl.BlockSpec((1, 9), lambda c, i: (0, 0)),
            pl.BlockSpec((9, 32), lambda c, i: (0, 0)),
            pl.BlockSpec((1, 32), lambda c, i: (0, 0)),
        ],
        out_specs=pl.BlockSpec((1, _BE, 16), lambda c, i: (c, i, 0)),
        out_shape=jax.ShapeDtypeStruct((2, E_PAD, 16), jnp.float32),
    )(edge_attr, W2a, b2a.reshape(1, 9), W2b, b2b.reshape(1, 32))


CHUNK = 256
RPW = EPW // 128


def _sc_aggr_body(h2_hbm, ea2_hbm, src_hbm, dst_hbm, out_hbm,
                  aggr_sh, sidx, didx, rows, eab,
                  lsem0, lsem1, dsem0, dsem1, gsem0, gsem1, ssem0, ssem1):
    c = lax.axis_index("c")
    s = lax.axis_index("s")
    lsem = (lsem0, lsem1)
    dsem = (dsem0, dsem1)
    gsem = (gsem0, gsem1)
    ssem = (ssem0, ssem1)

    @plsc.parallel_loop(0, 512)
    def _z(i):
        rows[i] = jnp.zeros((16,), jnp.float32)
    for k in range(12):
        pltpu.sync_copy(rows, aggr_sh.at[pl.ds(s * ZROWS + k * 512, 512)])
    pltpu.sync_copy(rows.at[pl.ds(0, 112)],
                    aggr_sh.at[pl.ds(s * ZROWS + 6144, 112)])
    plsc.subcore_barrier()

    def fire_loads(g, b):
        rbase = s * RPW + g * 2
        ebase = c * E_PAD + s * EPW + g * CHUNK
        pltpu.async_copy(src_hbm.at[pl.ds(c * ER + rbase, 2)],
                         sidx.at[pl.ds(b * 2, 2)], lsem[b])
        pltpu.async_copy(ea2_hbm.at[pl.ds(ebase, CHUNK)],
                         eab.at[pl.ds(b * CHUNK, CHUNK)], lsem[b])

    def fire_didx(g, b):
        rbase = s * RPW + g * 2
        pltpu.async_copy(dst_hbm.at[pl.ds(rbase, 2)],
                         didx.at[pl.ds(b * 2, 2)], dsem[b])

    def wait_loads(b):
        pltpu.make_async_copy(src_hbm.at[pl.ds(0, 2)],
                              sidx.at[pl.ds(b * 2, 2)], lsem[b]).wait()
        pltpu.make_async_copy(ea2_hbm.at[pl.ds(0, CHUNK)],
                              eab.at[pl.ds(b * CHUNK, CHUNK)], lsem[b]).wait()

    def wait_didx(b):
        pltpu.make_async_copy(dst_hbm.at[pl.ds(0, 2)],
                              didx.at[pl.ds(b * 2, 2)], dsem[b]).wait()

    def fire_gathers(b):
        for j in range(2):
            pltpu.async_copy(h2_hbm.at[sidx.at[b * 2 + j]],
                             rows.at[pl.ds((b * 2 + j) * 128, 128)], gsem[b])

    def wait_gathers(b):
        for j in range(2):
            pltpu.make_async_copy(
                h2_hbm.at[pl.ds(0, 128)],
                rows.at[pl.ds((b * 2 + j) * 128, 128)], gsem[b]).wait()

    def fire_scatters(b):
        for j in range(2):
            pltpu.async_copy(rows.at[pl.ds((b * 2 + j) * 128, 128)],
                             aggr_sh.at[didx.at[b * 2 + j]], ssem[b], add=True)

    def wait_scatters(b):
        for j in range(2):
            pltpu.make_async_copy(rows.at[pl.ds((b * 2 + j) * 128, 128)],
                                  aggr_sh.at[pl.ds(0, 128)], ssem[b]).wait()

    def compute(b):
        @plsc.parallel_loop(0, CHUNK, unroll=8)
        def _cmp(i):
            k = b * CHUNK + i
            rows[k] = jnp.maximum(rows[k] + eab[k], 0.0)

    fire_loads(0, 0)
    fire_didx(0, 0)
    wait_loads(0)
    fire_gathers(0)
    fire_loads(1, 1)

    def step(g, b, first, fire_next_gather, fire_next_loads):
        if fire_next_gather:
            wait_loads(1 - b)
        if first:
            wait_scatters(1 - b)
        if fire_next_gather:
            fire_didx(g + 1, 1 - b)
            fire_gathers(1 - b)
        wait_gathers(b)
        compute(b)
        wait_didx(b)
        fire_scatters(b)
        if fire_next_loads:
            fire_loads(g + 2, b)

    def body(gg, carry):
        g0 = gg * 2

        @pl.when(gg > 0)
        def _():
            wait_scatters(1)
        wait_loads(1)
        fire_didx(g0 + 1, 1)
        fire_gathers(1)
        wait_gathers(0)
        compute(0)
        wait_didx(0)
        fire_scatters(0)
        fire_loads(g0 + 2, 0)

        step(g0 + 1, 1, True, True, True)
        return carry

    lax.fori_loop(0, CHUNKS // 2 - 1, body, 0)
    step(CHUNKS - 2, 0, True, True, False)
    step(CHUNKS - 1, 1, True, False, False)
    wait_scatters(1)

    plsc.subcore_barrier()
    for k in range(12):
        off = s * ZROWS + k * 512
        pltpu.sync_copy(aggr_sh.at[pl.ds(off, 512)],
                        out_hbm.at[pl.ds(c * NT + off, 512)])
    off = s * ZROWS + 6144
    pltpu.sync_copy(aggr_sh.at[pl.ds(off, 112)],
                    out_hbm.at[pl.ds(c * NT + off, 112)])


def _sc_aggregate(h2_flat, ea2_flat, src2, dstp):
    mesh = plsc.VectorSubcoreMesh(core_axis_name="c", subcore_axis_name="s")
    f = pl.kernel(
        _sc_aggr_body,
        out_type=jax.ShapeDtypeStruct((2 * NT, 16), jnp.float32),
        mesh=mesh,
        compiler_params=pltpu.CompilerParams(use_tc_tiling_on_sc=False),
        scratch_types=[
            pltpu.VMEM_SHARED((NT, 16), jnp.float32),
            pltpu.VMEM((4, 128), jnp.int32),
            pltpu.VMEM((4, 128), jnp.int32),
            pltpu.VMEM((512, 16), jnp.float32),
            pltpu.VMEM((512, 16), jnp.float32),
        ] + [pltpu.SemaphoreType.DMA] * 8,
    )
    return f(h2_flat, ea2_flat, src2, dstp)


def _layer_mlp_body(h0_ref, h1_ref, a0_ref, a1_ref, Wc1_ref, bc1_ref,
                    Wc2_ref, bc2_ref, z_ref, sums_ref):
    i = pl.program_id(0)
    z0 = jnp.concatenate(
        [h0_ref[...] + a0_ref[...], h1_ref[...] + a1_ref[...]], axis=1)
    t = jnp.maximum(
        jnp.dot(z0, Wc1_ref[...], preferred_element_type=jnp.float32)
        + bc1_ref[...], 0.0)
    z = jnp.dot(t, Wc2_ref[...], preferred_element_type=jnp.float32) + bc2_ref[...]
    z_ref[...] = z
    blk = jnp.concatenate(
        [jnp.sum(z, axis=0, keepdims=True),
         jnp.sum(z * z, axis=0, keepdims=True)], axis=0)

    @pl.when(i == 0)
    def _():
        sums_ref[...] = blk

    @pl.when(i != 0)
    def _():
        sums_ref[...] += blk


def _layer_mlp(h2_flat, aggr_flat, Wc1, bc1, Wc2, bc2):
    half = pl.BlockSpec((_BN, 16), lambda i: (i, 0))
    return pl.pallas_call(
        _layer_mlp_body,
        grid=(N // _BN,),
        in_specs=[
            half, half, half, half,
            pl.BlockSpec((32, 64), lambda i: (0, 0)),
            pl.BlockSpec((1, 64), lambda i: (0, 0)),
            pl.BlockSpec((64, 32), lambda i: (0, 0)),
            pl.BlockSpec((1, 32), lambda i: (0, 0)),
        ],
        out_specs=[
            pl.BlockSpec((_BN, 32), lambda i: (i, 0)),
            pl.BlockSpec((2, 32), lambda i: (0, 0)),
        ],
        out_shape=[
            jax.ShapeDtypeStruct((N, 32), jnp.float32),
            jax.ShapeDtypeStruct((2, 32), jnp.float32),
        ],
    )(h2_flat[:N], h2_flat[NT:NT + N], aggr_flat[:N], aggr_flat[NT:NT + N],
      Wc1, bc1.reshape(1, 64), Wc2, bc2.reshape(1, 32))


def _bn_mid_body(z_ref, sc_ref, sh_ref, out_ref):
    c = pl.program_id(0)
    val = jnp.maximum(z_ref[...] * sc_ref[...] + sh_ref[...], 0.0)
    half = jnp.where(c == 0, val[:, :16], val[:, 16:])
    out_ref[...] = half[None]


def _bn_mid(z, scale, shift):
    return pl.pallas_call(
        _bn_mid_body,
        grid=(2, N // _BN),
        in_specs=[
            pl.BlockSpec((_BN, 32), lambda c, i: (i, 0)),
            pl.BlockSpec((1, 32), lambda c, i: (0, 0)),
            pl.BlockSpec((1, 32), lambda c, i: (0, 0)),
        ],
        out_specs=pl.BlockSpec((1, _BN, 16), lambda c, i: (c, i, 0)),
        out_shape=jax.ShapeDtypeStruct((2, NT, 16), jnp.float32),
    )(z, scale, shift)


def _bn_final_body(z_ref, sc_ref, sh_ref, out_ref):
    out_ref[...] = z_ref[...] * sc_ref[...] + sh_ref[...]


def _bn_final(z, scale, shift):
    return pl.pallas_call(
        _bn_final_body,
        grid=(N // _BN,),
        in_specs=[
            pl.BlockSpec((_BN, 32), lambda i: (i, 0)),
            pl.BlockSpec((1, 32), lambda i: (0, 0)),
            pl.BlockSpec((1, 32), lambda i: (0, 0)),
        ],
        out_specs=pl.BlockSpec((_BN, 32), lambda i: (i, 0)),
        out_shape=jax.ShapeDtypeStruct((N, 32), jnp.float32),
    )(z, scale, shift)


def kernel(x, edge_attr, edge_index, W1a, b1a, W1b, b1b, W2a, b2a, W2b, b2b,
           Wc1, bc1, Wc2, bc2, gamma, beta):
    src = edge_index[0]
    dst = edge_index[1]

    pad = N + (jnp.arange(E_PAD - E, dtype=jnp.int32) % 16)
    srcp = jnp.concatenate([src, pad])
    dstp = jnp.concatenate([dst, pad]).reshape(ER, 128)
    src2 = (srcp[None, :]
            + jnp.array([0, NT], dtype=jnp.int32)[:, None]).reshape(2 * ER, 128)

    h2 = _node_mlp(x, W1a, b1a, W1b, b1b).reshape(2 * NT, 16)
    ea2 = _edge_mlp(edge_attr, W2a, b2a, W2b, b2b).reshape(2 * E_PAD, 16)

    for l in range(L):
        aggr = _sc_aggregate(h2, ea2, src2, dstp)
        z, sums = _layer_mlp(h2, aggr, Wc1[l], bc1[l], Wc2[l], bc2[l])
        mean = sums[0] / N
        var = sums[1] / N - mean * mean
        scale = (gamma[l] / jnp.sqrt(var + 1e-5)).reshape(1, 32)
        shift = (beta[l] - mean * scale[0]).reshape(1, 32)
        if l != L - 1:
            h2 = _bn_mid(z, scale, shift).reshape(2 * NT, 16)
        else:
            out = _bn_final(z, scale, shift)
    return out
